# Initial kernel scaffold; baseline (speedup 1.0000x reference)
import jax
import jax.numpy as jnp
from jax.experimental import pallas as pl


def kernel(x, edge_index, edge_attr, W1, b1, W2, b2):
    raise NotImplementedError("write your pallas kernel here")


# speedup vs baseline: 2.1432x; 2.1432x over previous
import functools

import jax
import jax.numpy as jnp
import numpy as np
from jax import lax
from jax.experimental import pallas as pl
from jax.experimental.pallas import tpu as pltpu
from jax.experimental.pallas import tpu_sc as plsc

N = 10000
E = 160000
D = 512
PN = 10240

NC, NS, L = 2, 16, 16
NW = NC * NS
EW = E // NW
B = 64
EWPAD = 5056
NBATCH = EWPAD // B
SCHUNK = 2048
SPANS = (SCHUNK,) * (PN // SCHUNK)
ACC_ROWS = SCHUNK + NS

_mesh = plsc.VectorSubcoreMesh(core_axis_name="c", subcore_axis_name="s")

f32 = jnp.float32
i32 = jnp.int32


def _zero16f():
    return jnp.zeros((L,), f32)


def _zero16i():
    return jnp.zeros((L,), i32)


def _deg_body(col_hbm, ew_hbm, d1_hbm, d2_hbm,
              colb, ewb, idxb, idxt, valt, onesb, zb, d1acc, d2acc):
    c = lax.axis_index("c")
    s = lax.axis_index("s")
    wid = c * NS + s
    e0 = wid * EW
    pltpu.sync_copy(col_hbm.at[pl.ds(e0, EW)], colb.at[pl.ds(0, EW)])
    pltpu.sync_copy(ew_hbm.at[pl.ds(e0, EW)], ewb.at[pl.ds(0, EW)])

    def _fill(i, _):
        off = pl.multiple_of(i * L, L)
        zb[pl.ds(off, L)] = _zero16f()
        return 0
    lax.fori_loop(0, 640 // L, _fill, 0)
    for g in range(128 // L):
        onesb[pl.ds(g * L, L)] = jnp.ones((L,), f32)

    pltpu.sync_copy(zb, d1acc.at[pl.ds(s * 640, 640)])
    pltpu.sync_copy(zb, d2acc.at[pl.ds(s * 640, 640)])
    plsc.subcore_barrier()

    def _batch(k, _):
        off = pl.multiple_of(k * 128, 128)
        for g in range(128 // L):
            idxb[pl.ds(g * L, L)] = colb[pl.ds(off + g * L, L)]
        pltpu.sync_copy(ewb.at[pl.ds(off, 128)], d1acc.at[idxb], add=True)
        pltpu.sync_copy(onesb, d2acc.at[idxb], add=True)
        return 0
    lax.fori_loop(0, EW // 128, _batch, 0)

    toff = (EW // 128) * 128
    valid = lax.iota(i32, L) < (EW - toff)
    col16 = colb[pl.ds(toff, L)]
    ew16 = ewb[pl.ds(toff, L)]
    idxt[pl.ds(0, L)] = jnp.where(valid, col16, 0)
    valt[pl.ds(0, L)] = jnp.where(valid, ew16, 0.0)
    pltpu.sync_copy(valt, d1acc.at[idxt], add=True)
    valt[pl.ds(0, L)] = jnp.where(valid, jnp.ones((L,), f32), 0.0)
    pltpu.sync_copy(valt, d2acc.at[idxt], add=True)

    plsc.subcore_barrier()

    @pl.when(s == 0)
    def _():
        pltpu.sync_copy(d1acc, d1_hbm.at[c])
        pltpu.sync_copy(d2acc, d2_hbm.at[c])


def _degrees(col, ew):
    return pl.kernel(
        _deg_body,
        out_type=[jax.ShapeDtypeStruct((NC, PN), f32),
                  jax.ShapeDtypeStruct((NC, PN), f32)],
        mesh=_mesh,
        scratch_types=[
            pltpu.VMEM((EWPAD,), i32),
            pltpu.VMEM((EWPAD,), f32),
            pltpu.VMEM((128,), i32),
            pltpu.VMEM((L,), i32),
            pltpu.VMEM((L,), f32),
            pltpu.VMEM((128,), f32),
            pltpu.VMEM((640,), f32),
            pltpu.VMEM_SHARED((PN,), f32),
            pltpu.VMEM_SHARED((PN,), f32),
        ],
    )(col, ew)


def _splat_lane(v, lane):
    idx = jnp.zeros((L,), i32) + lane
    return lax.gather(
        v, idx[:, None],
        dimension_numbers=lax.GatherDimensionNumbers(
            offset_dims=(), collapsed_slice_dims=(0,), start_index_map=(0,)),
        slice_sizes=(1,),
        mode=lax.GatherScatterMode.PROMISE_IN_BOUNDS)


DW = 128
XP = D // DW


def _spmm_body(has_ew, *refs):
    if has_ew:
        (y_hbm, row_hbm, col_hbm, ew_hbm, parts_hbm,
         rowb, colb, ewb, ridx, cidx, gbuf, zb, acc, sem) = refs
    else:
        (y_hbm, row_hbm, col_hbm, parts_hbm,
         rowb, colb, ridx, cidx, gbuf, zb, acc, sem) = refs

    c = lax.axis_index("c")
    s = lax.axis_index("s")
    wid = c * NS + s
    e0 = wid * EW
    pltpu.sync_copy(row_hbm.at[pl.ds(e0, EW)], rowb.at[pl.ds(0, EW)])
    pltpu.sync_copy(col_hbm.at[pl.ds(e0, EW)], colb.at[pl.ds(0, EW)])
    if has_ew:
        pltpu.sync_copy(ew_hbm.at[pl.ds(e0, EW)], ewb.at[pl.ds(0, EW)])

    def _zrow(i, _):
        for u in range(XP):
            for j in range(DW // L):
                zb[i, u, pl.ds(j * L, L)] = _zero16f()
        return 0
    lax.fori_loop(0, 16, _zrow, 0)

    lanes = lax.iota(i32, L)
    dumpv = jnp.zeros((L,), i32) + (SCHUNK + s)

    for ci, span in enumerate(SPANS):
        lo = ci * SCHUNK
        rpw = span // NS

        for t in range(rpw // 16):
            pltpu.sync_copy(zb, acc.at[pl.ds(s * rpw + t * 16, 16)])
        plsc.subcore_barrier()

        def _batch(k, _):
            off = pl.multiple_of(k * B, B)
            for g in range(B // L):
                col16 = colb[pl.ds(off + g * L, L)]
                row16 = rowb[pl.ds(off + g * L, L)]
                valid = (lanes + (off + g * L)) < EW
                m = valid & (col16 >= lo) & (col16 < lo + SCHUNK)
                ridx[pl.ds(g * L, L)] = jnp.where(valid, row16, 0)
                cidx[pl.ds(g * L, L)] = jnp.where(m, col16 - lo, dumpv)
            pltpu.async_copy(y_hbm.at[ridx], gbuf, sem).wait()
            if has_ew:
                def _srow(e, _):
                    o2 = pl.multiple_of(off + (e // L) * L, L)
                    ew16 = ewb[pl.ds(o2, L)]
                    spl = _splat_lane(ew16, e % L)
                    for u in range(XP):
                        for j in range(DW // L):
                            gbuf[e, u, pl.ds(j * L, L)] = \
                                gbuf[e, u, pl.ds(j * L, L)] * spl
                    return 0
                lax.fori_loop(0, B, _srow, 0)
            pltpu.sync_copy(gbuf, acc.at[cidx], add=True)
            return 0
        lax.fori_loop(0, NBATCH, _batch, 0)
        plsc.subcore_barrier()

        for t in range(rpw // 16):
            r0 = s * rpw + t * 16
            pltpu.sync_copy(acc.at[pl.ds(r0, 16)],
                            parts_hbm.at[c, pl.ds(lo + r0, 16)])
        plsc.subcore_barrier()


def _spmm(y, row, col, ew=None):
    has_ew = ew is not None
    scratch = [
        pltpu.VMEM((EWPAD,), i32),
        pltpu.VMEM((EWPAD,), i32),
    ]
    if has_ew:
        scratch.append(pltpu.VMEM((EWPAD,), f32))
    scratch += [
        pltpu.VMEM((B,), i32),
        pltpu.VMEM((B,), i32),
        pltpu.VMEM((B, XP, DW), f32),
        pltpu.VMEM((16, XP, DW), f32),
        pltpu.VMEM_SHARED((ACC_ROWS, XP, DW), f32),
        pltpu.SemaphoreType.DMA,
    ]
    args = (y.reshape(N, XP, DW), row, col)
    if has_ew:
        args = args + (ew,)
    parts4 = pl.kernel(
        functools.partial(_spmm_body, has_ew),
        out_type=jax.ShapeDtypeStruct((NC, PN, XP, DW), f32),
        mesh=_mesh,
        scratch_types=scratch,
    )(*args)
    return parts4.reshape(NC, PN, D)


def _dinv_body(d1_ref, d2_ref, o1_ref, o2_ref):
    o1_ref[...] = lax.rsqrt(d1_ref[0] + d1_ref[1] + 1.0)
    o2_ref[...] = lax.rsqrt(d2_ref[0] + d2_ref[1] + 1.0)


def _dinvs(d1p, d2p):
    return pl.pallas_call(
        _dinv_body,
        out_shape=[jax.ShapeDtypeStruct((PN,), f32),
                   jax.ShapeDtypeStruct((PN,), f32)],
    )(d1p, d2p)


BM = 1000


def _mm_scale_body(x_ref, w_ref, dv_ref, y_ref):
    y_ref[...] = jnp.dot(x_ref[...], w_ref[...],
                         preferred_element_type=f32) * dv_ref[...]


def _mm_scale(x, w, dv):
    return pl.pallas_call(
        _mm_scale_body,
        grid=(N // BM,),
        in_specs=[
            pl.BlockSpec((BM, D), lambda i: (i, 0)),
            pl.BlockSpec((D, D), lambda i: (0, 0)),
            pl.BlockSpec((BM, 1), lambda i: (i, 0)),
        ],
        out_specs=pl.BlockSpec((BM, D), lambda i: (i, 0)),
        out_shape=jax.ShapeDtypeStruct((N, D), f32),
    )(x, w, dv)


def _mid_body(p_ref, y1_ref, dv1_ref, b1_ref, w2_ref, dv2_ref, y2_ref):
    pre = (p_ref[0] + p_ref[1] + y1_ref[...]) * dv1_ref[...] + b1_ref[...]
    h = jnp.maximum(pre, 0.0)
    y2_ref[...] = jnp.dot(h, w2_ref[...], preferred_element_type=f32) * dv2_ref[...]


def _mid(p1, y1, dv1, b1, w2, dv2):
    return pl.pallas_call(
        _mid_body,
        grid=(N // BM,),
        in_specs=[
            pl.BlockSpec((NC, BM, D), lambda i: (0, i, 0)),
            pl.BlockSpec((BM, D), lambda i: (i, 0)),
            pl.BlockSpec((BM, 1), lambda i: (i, 0)),
            pl.BlockSpec((1, D), lambda i: (0, 0)),
            pl.BlockSpec((D, D), lambda i: (0, 0)),
            pl.BlockSpec((BM, 1), lambda i: (i, 0)),
        ],
        out_specs=pl.BlockSpec((BM, D), lambda i: (i, 0)),
        out_shape=jax.ShapeDtypeStruct((N, D), f32),
    )(p1, y1, dv1, b1, w2, dv2)


def _final_body(p_ref, y2_ref, dv2_ref, b2_ref, o_ref):
    o_ref[...] = (p_ref[0] + p_ref[1] + y2_ref[...]) * dv2_ref[...] + b2_ref[...]


def _final(p2, y2, dv2, b2):
    return pl.pallas_call(
        _final_body,
        grid=(N // BM,),
        in_specs=[
            pl.BlockSpec((NC, BM, D), lambda i: (0, i, 0)),
            pl.BlockSpec((BM, D), lambda i: (i, 0)),
            pl.BlockSpec((BM, 1), lambda i: (i, 0)),
            pl.BlockSpec((1, D), lambda i: (0, 0)),
        ],
        out_specs=pl.BlockSpec((BM, D), lambda i: (i, 0)),
        out_shape=jax.ShapeDtypeStruct((N, D), f32),
    )(p2, y2, dv2, b2)


@jax.jit
def kernel(x, edge_index, edge_attr, W1, b1, W2, b2):
    row = edge_index[0]
    col = edge_index[1]

    d1p, d2p = _degrees(col, edge_attr)
    dinv1, dinv2 = _dinvs(d1p, d2p)
    dv1 = dinv1[:N].reshape(N, 1)
    dv2 = dinv2[:N].reshape(N, 1)

    y1 = _mm_scale(x, W1, dv1)
    p1 = _spmm(y1, row, col, edge_attr)
    y2 = _mid(p1, y1, dv1, b1.reshape(1, D), W2, dv2)
    p2 = _spmm(y2, row, col)
    return _final(p2, y2, dv2, b2.reshape(1, D))
